# Initial kernel scaffold; baseline (speedup 1.0000x reference)
import functools

import jax
import jax.numpy as jnp
from jax import lax
from jax.experimental import pallas as pl
from jax.experimental.pallas import tpu as pltpu
from jax.experimental.pallas import tpu_sc as plsc

BATCH = 16384
HIST = 50
LENGTH = 64
B = BATCH * HIST
NW = 32
BPW = B // NW
G = 128
NG = BPW // G

_mesh = plsc.VectorSubcoreMesh(core_axis_name="c", subcore_axis_name="s")


@functools.partial(
    pl.kernel,
    mesh=_mesh,
    out_type=[
        jax.ShapeDtypeStruct((B, LENGTH), jnp.float32),
        jax.ShapeDtypeStruct((B, LENGTH), jnp.float32),
        jax.ShapeDtypeStruct((B, 2), jnp.float32),
    ],
    scratch_types=[
        pltpu.VMEM((BPW,), jnp.int32),
        pltpu.VMEM((G, LENGTH), jnp.float32),
        pltpu.VMEM((G, LENGTH), jnp.float32),
        pltpu.VMEM((G, 2), jnp.float32),
        pltpu.SemaphoreType.DMA,
    ],
)
def _gather3(idx_hbm, row_hbm, col_hbm, dir_hbm,
             row_out, col_out, dir_out,
             idx_v, row_v, col_v, dir_v, sem):
    wid = lax.axis_index("s") * 2 + lax.axis_index("c")
    base = wid * BPW
    pltpu.sync_copy(idx_hbm.at[pl.ds(base, BPW)], idx_v)

    def body(g, carry):
        off = pl.multiple_of(g * G, G)
        idxs = idx_v.at[pl.ds(off, G)]
        pltpu.async_copy(row_hbm.at[idxs], row_v, sem).wait()
        pltpu.async_copy(col_hbm.at[idxs], col_v, sem).wait()
        pltpu.async_copy(dir_hbm.at[idxs], dir_v, sem).wait()
        pltpu.sync_copy(row_v, row_out.at[pl.ds(base + off, G)])
        pltpu.sync_copy(col_v, col_out.at[pl.ds(base + off, G)])
        pltpu.sync_copy(dir_v, dir_out.at[pl.ds(base + off, G)])
        return carry

    lax.fori_loop(0, NG, body, 0)


def kernel(input, row_table, col_table, dir_table):
    idx = input.reshape(-1).astype(jnp.int32)
    row, col, dir_ = _gather3(idx, row_table, col_table, dir_table)
    return (
        row.reshape(BATCH, HIST, LENGTH),
        col.reshape(BATCH, HIST, LENGTH),
        dir_.reshape(BATCH, HIST, 2),
    )


# speedup vs baseline: 3.3884x; 3.3884x over previous
import functools

import jax
import jax.numpy as jnp
from jax import lax
from jax.experimental import pallas as pl
from jax.experimental.pallas import tpu as pltpu
from jax.experimental.pallas import tpu_sc as plsc

BATCH = 16384
HIST = 50
LENGTH = 64
B = BATCH * HIST
NW = 32
BPW = B // NW
G = 128
NG = BPW // G

_mesh = plsc.VectorSubcoreMesh(core_axis_name="c", subcore_axis_name="s")


@functools.partial(
    pl.kernel,
    mesh=_mesh,
    out_type=[
        jax.ShapeDtypeStruct((B, LENGTH), jnp.float32),
        jax.ShapeDtypeStruct((B, LENGTH), jnp.float32),
        jax.ShapeDtypeStruct((B, 2), jnp.float32),
    ],
    scratch_types=[
        pltpu.VMEM((NG, G), jnp.int32),
        pltpu.VMEM((G, LENGTH), jnp.float32),
        pltpu.VMEM((G, LENGTH), jnp.float32),
        pltpu.VMEM((G, 16), jnp.float32),
        pltpu.SemaphoreType.DMA,
    ],
    compiler_params=pltpu.CompilerParams(use_tc_tiling_on_sc=False),
)
def _gather3(idx_hbm, row_hbm, col_hbm, dir_hbm,
             row_out, col_out, dir_out,
             idx_v, row_v, col_v, dir_v, sem):
    wid = lax.axis_index("s") * 2 + lax.axis_index("c")
    base = wid * BPW
    pltpu.sync_copy(idx_hbm.at[wid], idx_v)

    def body(g, carry):
        off = pl.multiple_of(g * G, G)
        idxs = idx_v.at[g]
        pltpu.async_copy(row_hbm.at[idxs], row_v, sem).wait()
        pltpu.async_copy(col_hbm.at[idxs], col_v, sem).wait()
        pltpu.async_copy(dir_hbm.at[idxs], dir_v, sem).wait()
        pltpu.sync_copy(row_v, row_out.at[pl.ds(base + off, G)])
        pltpu.sync_copy(col_v, col_out.at[pl.ds(base + off, G)])
        pltpu.sync_copy(dir_v.at[:, pl.ds(0, 2)], dir_out.at[pl.ds(base + off, G)])
        return carry

    lax.fori_loop(0, NG, body, 0)


def kernel(input, row_table, col_table, dir_table):
    idx = input.reshape(NW, NG, G).astype(jnp.int32)
    dir_wide = jnp.pad(dir_table, ((0, 0), (0, 14)))
    row, col, dir_ = _gather3(idx, row_table, col_table, dir_wide)
    return (
        row.reshape(BATCH, HIST, LENGTH),
        col.reshape(BATCH, HIST, LENGTH),
        dir_.reshape(BATCH, HIST, 2),
    )
